# baseline (speedup 1.0000x reference)
import functools

import jax
import jax.numpy as jnp
from jax import lax
from jax.experimental import pallas as pl
from jax.experimental.pallas import tpu as pltpu
from jax.experimental.pallas import tpu_sc as plsc

_B, _S, _WL = 32, 512, 16
_WDIM, _CDIM, _F, _W = 128, 64, 64, 5
_HID = _WDIM + _F
_NWORDS = _B * _S
_NCHARS = _NWORDS * _WL
_NPOS = _WL - _W + 1

_NC, _NS = 2, 16
_NWK = _NC * _NS
_CHUNK = 128
_WCH = _NWORDS // (_NWK * _CHUNK)
_CCH = _NCHARS // (_NWK * _CHUNK)


_CW = _CDIM // 2


def _sc_gather_body(wtbl, widx, ctbl, cidx, wout, cout,
                    widx_v, cidx_v, wbuf, cbuf, sem):
    wid = lax.axis_index("s") * _NC + lax.axis_index("c")
    pltpu.sync_copy(widx.at[wid], widx_v)
    pltpu.sync_copy(cidx.at[wid], cidx_v)
    wbase = wid * (_WCH * _CHUNK)
    cbase = wid * (_CCH * _CHUNK)

    def wbody(j, carry):
        pltpu.async_copy(wtbl.at[widx_v.at[j]], wbuf, sem).wait()
        pltpu.sync_copy(wbuf, wout.at[pl.ds(wbase + j * _CHUNK, _CHUNK)])
        return carry

    lax.fori_loop(0, _WCH, wbody, 0)

    def cbody(j, carry):
        pltpu.async_copy(ctbl.at[cidx_v.at[j]], cbuf, sem).wait()
        pltpu.sync_copy(cbuf, cout.at[pl.ds(cbase + j * _CHUNK, _CHUNK)])
        return carry

    lax.fori_loop(0, _CCH, cbody, 0)


@functools.cache
def _get_sc_gather():
    return pl.kernel(
        _sc_gather_body,
        out_type=[
            jax.ShapeDtypeStruct((_NWORDS, _WDIM), jnp.float32),
            jax.ShapeDtypeStruct((_NCHARS, _CDIM), jnp.bfloat16),
        ],
        mesh=plsc.VectorSubcoreMesh(core_axis_name="c", subcore_axis_name="s",
                                    num_cores=_NC, num_subcores=_NS),
        scratch_types=[
            pltpu.VMEM((_WCH, _CHUNK), jnp.int32),
            pltpu.VMEM((_CCH, _CHUNK), jnp.int32),
            pltpu.VMEM((_CHUNK, _WDIM), jnp.float32),
            pltpu.VMEM((_CHUNK, _CDIM), jnp.bfloat16),
            pltpu.SemaphoreType.DMA,
        ],
        compiler_params=pltpu.CompilerParams(use_tc_tiling_on_sc=False),
    )


_M = 512


def _tc_dense_body(ce_ref, x_ref, wr_ref, unk_ref, pwt_ref, wf_ref, cb_ref,
                   gwt0_ref, gb0_ref, twt0_ref, tb0_ref,
                   gwt1_ref, gb1_ref, twt1_ref, tb1_ref, o_ref):
    f32 = jnp.float32
    z2 = jnp.dot(ce_ref[...], wf_ref[...], preferred_element_type=f32)
    v = z2[:, 0:_F]
    for k in range(1, _W):
        v = v + jnp.roll(z2[:, k * _F:(k + 1) * _F], -k, axis=0)
    r = jnp.maximum(v + cb_ref[...], 0.0)
    m2 = jnp.maximum(r, jnp.roll(r, -1, axis=0))
    m4 = jnp.maximum(m2, jnp.roll(m2, -2, axis=0))
    m8 = jnp.maximum(m4, jnp.roll(m4, -4, axis=0))
    m12 = jnp.maximum(m8, jnp.roll(m4, -8, axis=0))
    cm = m12.reshape(_M, _WL, _F)[:, 0, :]
    mask = x_ref[...] == 1
    emb = jnp.where(mask, unk_ref[...], wr_ref[...])
    p = jnp.dot(emb, pwt_ref[...], preferred_element_type=f32)
    h = jnp.concatenate([p, cm], axis=1)
    for gwt, gb, twt, tb in ((gwt0_ref, gb0_ref, twt0_ref, tb0_ref),
                             (gwt1_ref, gb1_ref, twt1_ref, tb1_ref)):
        g = jax.nn.sigmoid(jnp.dot(h, gwt[...], preferred_element_type=f32)
                           + gb[...])
        t = jnp.maximum(jnp.dot(h, twt[...], preferred_element_type=f32)
                        + tb[...], 0.0)
        h = g * t + (1.0 - g) * h
    o_ref[...] = h


def _full(shape):
    return pl.BlockSpec(shape, lambda i: (0, 0))


_tc_dense = pl.pallas_call(
    _tc_dense_body,
    grid=(_NWORDS // _M,),
    in_specs=[
        pl.BlockSpec((_M * _WL, _CDIM), lambda i: (i, 0)),
        pl.BlockSpec((_M, 1), lambda i: (i, 0)),
        pl.BlockSpec((_M, _WDIM), lambda i: (i, 0)),
        _full((1, _WDIM)),
        _full((_WDIM, _WDIM)),
        _full((_CDIM, _W * _F)),
        _full((1, _F)),
        _full((_HID, _HID)), _full((1, _HID)),
        _full((_HID, _HID)), _full((1, _HID)),
        _full((_HID, _HID)), _full((1, _HID)),
        _full((_HID, _HID)), _full((1, _HID)),
    ],
    out_specs=pl.BlockSpec((_M, _HID), lambda i: (i, 0)),
    out_shape=jax.ShapeDtypeStruct((_NWORDS, _HID), jnp.float32),
)


def _conv_taps(conv_w):
    return jnp.transpose(conv_w, (1, 2, 0)).reshape(_CDIM, _W * _F)


def kernel(x, c, word_table, unk_emb, proj_w, char_table, conv_w, conv_b,
           tw0, tb0, tw1, tb1, gw0, gb0, gw1, gb1):
    xf = x.astype(jnp.int32).reshape(-1)
    cf = c.astype(jnp.int32).reshape(-1)
    wrows, ce = _get_sc_gather()(
        word_table, xf.reshape(_NWK, _WCH, _CHUNK),
        char_table.astype(jnp.bfloat16), cf.reshape(_NWK, _CCH, _CHUNK))
    out = _tc_dense(
        ce, xf.reshape(-1, 1), wrows, unk_emb, proj_w.T,
        _conv_taps(conv_w).astype(jnp.bfloat16), conv_b.reshape(1, _F),
        gw0.T, gb0.reshape(1, _HID), tw0.T, tb0.reshape(1, _HID),
        gw1.T, gb1.reshape(1, _HID), tw1.T, tb1.reshape(1, _HID))
    return out.reshape(_B, _S, _HID)


# speedup vs baseline: 1.6615x; 1.6615x over previous
import functools

import jax
import jax.numpy as jnp
from jax import lax
from jax.experimental import pallas as pl
from jax.experimental.pallas import tpu as pltpu
from jax.experimental.pallas import tpu_sc as plsc

_B, _S, _WL = 32, 512, 16
_WDIM, _CDIM, _F, _W = 128, 64, 64, 5
_HID = _WDIM + _F
_NWORDS = _B * _S
_NCHARS = _NWORDS * _WL
_NPOS = _WL - _W + 1

_NC, _NS = 2, 16
_NWK = _NC * _NS
_CHUNK = 128
_WCH = _NWORDS // (_NWK * _CHUNK)
_CCH = _NCHARS // (_NWK * _CHUNK)


def _sc_gather_body(wtbl, widx, ctbl, cidx, wout, cout,
                    widx_v, cidx_v, wbuf, cbufs, wsem, cgs, css):
    wid = lax.axis_index("s") * _NC + lax.axis_index("c")
    pltpu.sync_copy(widx.at[wid], widx_v)
    pltpu.sync_copy(cidx.at[wid], cidx_v)
    wbase = wid * (_WCH * _CHUNK)
    cbase = wid * (_CCH * _CHUNK)

    def cgather(j, b):
        return pltpu.make_async_copy(ctbl.at[cidx_v.at[j]], cbufs.at[b],
                                     cgs.at[b])

    def cstore(j, b):
        return pltpu.make_async_copy(
            cbufs.at[b], cout.at[pl.ds(cbase + j * _CHUNK, _CHUNK)], css.at[b])

    ngrp = _CCH // 4
    for b in range(4):
        cgather(b, b).start()
    for b in range(4):
        cgather(b, b).wait()
        cstore(b, b).start()
    for b in range(4):
        cgather(4 + b, 4 + b).start()

    def grp(g, carry):
        cur = 4 * (g % 2)
        oth = 4 * ((g + 1) % 2)
        for b in range(4):
            cstore(4 * (g - 1) + b, oth + b).wait()
            cgather(4 * (g + 1) + b, oth + b).start()
        for b in range(4):
            cgather(4 * g + b, cur + b).wait()
            cstore(4 * g + b, cur + b).start()
        return carry

    lax.fori_loop(1, ngrp - 1, grp, 0)

    glast = ngrp - 1
    for b in range(4):
        cgather(4 * glast + b, 4 + b).wait()
        cstore(4 * glast + b, 4 + b).start()
    for b in range(4):
        cstore(4 * (glast - 1) + b, b).wait()
        cstore(4 * glast + b, 4 + b).wait()

    def wbody(j, carry):
        pltpu.async_copy(wtbl.at[widx_v.at[j]], wbuf, wsem).wait()
        pltpu.sync_copy(wbuf, wout.at[pl.ds(wbase + j * _CHUNK, _CHUNK)])
        return carry

    lax.fori_loop(0, _WCH, wbody, 0)


@functools.cache
def _get_sc_gather():
    return pl.kernel(
        _sc_gather_body,
        out_type=[
            jax.ShapeDtypeStruct((_NWORDS, _WDIM), jnp.float32),
            jax.ShapeDtypeStruct((_NCHARS, _CDIM), jnp.float32),
        ],
        mesh=plsc.VectorSubcoreMesh(core_axis_name="c", subcore_axis_name="s",
                                    num_cores=_NC, num_subcores=_NS),
        scratch_types=[
            pltpu.VMEM((_WCH, _CHUNK), jnp.int32),
            pltpu.VMEM((_CCH, _CHUNK), jnp.int32),
            pltpu.VMEM((_CHUNK, _WDIM), jnp.float32),
            pltpu.VMEM((8, _CHUNK, _CDIM), jnp.float32),
            pltpu.SemaphoreType.DMA,
            pltpu.SemaphoreType.DMA((8,)),
            pltpu.SemaphoreType.DMA((8,)),
        ],
        compiler_params=pltpu.CompilerParams(use_tc_tiling_on_sc=False),
    )


_M = 512


def _tc_dense_body(ce_ref, x_ref, wr_ref, unk_ref, pwt_ref, wc_ref, cb_ref,
                   gwt0_ref, gb0_ref, twt0_ref, tb0_ref,
                   gwt1_ref, gb1_ref, twt1_ref, tb1_ref, o_ref):
    f32 = jnp.float32
    bf16 = jnp.bfloat16
    z = jnp.dot(ce_ref[...].astype(bf16), wc_ref[...],
                preferred_element_type=f32)
    cb = cb_ref[...]
    cm = jnp.maximum(z[:, 0:_F] + cb, 0.0)
    for t in range(1, _NPOS):
        cm = jnp.maximum(cm, jnp.maximum(z[:, t * _F:(t + 1) * _F] + cb, 0.0))
    mask = x_ref[...] == 1
    emb = jnp.where(mask, unk_ref[...], wr_ref[...])
    p = jnp.dot(emb.astype(bf16), pwt_ref[...], preferred_element_type=f32)
    h = jnp.concatenate([p, cm], axis=1)
    for gwt, gb, twt, tb in ((gwt0_ref, gb0_ref, twt0_ref, tb0_ref),
                             (gwt1_ref, gb1_ref, twt1_ref, tb1_ref)):
        hb = h.astype(bf16)
        g = jax.nn.sigmoid(jnp.dot(hb, gwt[...], preferred_element_type=f32)
                           + gb[...])
        t = jnp.maximum(jnp.dot(hb, twt[...], preferred_element_type=f32)
                        + tb[...], 0.0)
        h = g * t + (1.0 - g) * h
    o_ref[...] = h


def _full(shape):
    return pl.BlockSpec(shape, lambda i: (0, 0))


_tc_dense = pl.pallas_call(
    _tc_dense_body,
    grid=(_NWORDS // _M,),
    in_specs=[
        pl.BlockSpec((_M, _WL * _CDIM), lambda i: (i, 0)),
        pl.BlockSpec((_M, 1), lambda i: (i, 0)),
        pl.BlockSpec((_M, _WDIM), lambda i: (i, 0)),
        _full((1, _WDIM)),
        _full((_WDIM, _WDIM)),
        _full((_WL * _CDIM, _NPOS * _F)),
        _full((1, _F)),
        _full((_HID, _HID)), _full((1, _HID)),
        _full((_HID, _HID)), _full((1, _HID)),
        _full((_HID, _HID)), _full((1, _HID)),
        _full((_HID, _HID)), _full((1, _HID)),
    ],
    out_specs=pl.BlockSpec((_M, _HID), lambda i: (i, 0)),
    out_shape=jax.ShapeDtypeStruct((_NWORDS, _HID), jnp.float32),
)


def _conv_toeplitz(conv_w):
    kflat = jnp.transpose(conv_w, (2, 1, 0)).reshape(_W * _CDIM, _F)
    cols = [jnp.pad(kflat, ((_CDIM * t, _CDIM * (_NPOS - 1 - t)), (0, 0)))
            for t in range(_NPOS)]
    return jnp.concatenate(cols, axis=1)


def kernel(x, c, word_table, unk_emb, proj_w, char_table, conv_w, conv_b,
           tw0, tb0, tw1, tb1, gw0, gb0, gw1, gb1):
    bf16 = jnp.bfloat16
    xf = x.astype(jnp.int32).reshape(-1)
    cf = c.astype(jnp.int32).reshape(-1)
    wrows, crows = _get_sc_gather()(
        word_table, xf.reshape(_NWK, _WCH, _CHUNK),
        char_table, cf.reshape(_NWK, _CCH, _CHUNK))
    ce = crows.reshape(_NWORDS, _WL * _CDIM)
    out = _tc_dense(
        ce, xf.reshape(-1, 1), wrows, unk_emb, proj_w.T.astype(bf16),
        _conv_toeplitz(conv_w).astype(bf16), conv_b.reshape(1, _F),
        gw0.T.astype(bf16), gb0.reshape(1, _HID),
        tw0.T.astype(bf16), tb0.reshape(1, _HID),
        gw1.T.astype(bf16), gb1.reshape(1, _HID),
        tw1.T.astype(bf16), tb1.reshape(1, _HID))
    return out.reshape(_B, _S, _HID)
